# baseline (speedup 1.0000x reference)
import jax
import jax.numpy as jnp
from jax.experimental import pallas as pl
from jax.experimental.pallas import tpu as pltpu

VOCAB = 256
EMBED = 256
HIDDEN = 512
SEQ = 256
BATCH = 64
UNROLL = 16


def _lstm_kernel(x_col_ref, emb_ref, WihT_ref, WhhT_bf_ref, bias_ref,
                 WfcT_ref, bfc_ref, out_ref, W_ref):
    G = jnp.dot(emb_ref[:], WihT_ref[:],
                preferred_element_type=jnp.float32) + bias_ref[:]
    W_ref[pl.ds(0, VOCAB), :] = G.astype(jnp.bfloat16)
    W_ref[pl.ds(VOCAB, HIDDEN), :] = WhhT_bf_ref[:]

    def substep(t, h_bf, c):
        ids_t = x_col_ref[pl.ds(t * BATCH, BATCH), :]
        iota = jax.lax.broadcasted_iota(jnp.int32, (BATCH, VOCAB), 1)
        oh_t = (iota == ids_t).astype(jnp.bfloat16)
        a = jnp.concatenate([oh_t, h_bf], axis=1)
        gates = jnp.dot(a, W_ref[:], preferred_element_type=jnp.float32)

        def sig(v):
            return 0.5 * jnp.tanh(0.5 * v) + 0.5

        i = sig(gates[:, 0 * HIDDEN:1 * HIDDEN])
        f = sig(gates[:, 1 * HIDDEN:2 * HIDDEN])
        g = jnp.tanh(gates[:, 2 * HIDDEN:3 * HIDDEN])
        o = sig(gates[:, 3 * HIDDEN:4 * HIDDEN])
        c_new = f * c + i * g
        h_new = o * jnp.tanh(c_new)
        return h_new.astype(jnp.bfloat16), c_new

    def step(k, carry):
        h_bf, c = carry
        for u in range(UNROLL):
            h_bf, c = substep(UNROLL * k + u, h_bf, c)
        return (h_bf, c)

    h0 = jnp.zeros((BATCH, HIDDEN), jnp.bfloat16)
    c0 = jnp.zeros((BATCH, HIDDEN), jnp.float32)
    h_last, _ = jax.lax.fori_loop(0, SEQ // UNROLL, step, (h0, c0))

    out_ref[:] = (jnp.dot(h_last.astype(jnp.float32), WfcT_ref[:],
                          preferred_element_type=jnp.float32) + bfc_ref[:])


def kernel(x, emb, W_ih, W_hh, b_ih, b_hh, W_fc, b_fc):
    x_col = x.T.reshape(SEQ * BATCH, 1).astype(jnp.int32)
    WihT = W_ih.T
    WhhT_bf = W_hh.T.astype(jnp.bfloat16)
    WfcT = W_fc.T
    bias = (b_ih + b_hh).reshape(1, 4 * HIDDEN)
    bfc = b_fc.reshape(1, VOCAB)

    return pl.pallas_call(
        _lstm_kernel,
        out_shape=jax.ShapeDtypeStruct((BATCH, VOCAB), jnp.float32),
        scratch_shapes=[
            pltpu.VMEM((VOCAB + HIDDEN, 4 * HIDDEN), jnp.bfloat16)],
    )(x_col, emb, WihT, WhhT_bf, bias, WfcT, bfc)


# speedup vs baseline: 1.0170x; 1.0170x over previous
import jax
import jax.numpy as jnp
from jax.experimental import pallas as pl
from jax.experimental.pallas import tpu as pltpu

VOCAB = 256
EMBED = 256
HIDDEN = 512
SEQ = 256
BATCH = 64
UNROLL = 32


def _lstm_kernel(x_col_ref, emb_ref, WihT_ref, WhhT_bf_ref, bias_ref,
                 WfcT_ref, bfc_ref, out_ref, W_ref):
    G = jnp.dot(emb_ref[:], WihT_ref[:],
                preferred_element_type=jnp.float32) + bias_ref[:]
    W_ref[pl.ds(0, VOCAB), :] = G.astype(jnp.bfloat16)
    W_ref[pl.ds(VOCAB, HIDDEN), :] = WhhT_bf_ref[:]

    def substep(t, h_bf, c):
        ids_t = x_col_ref[pl.ds(t * BATCH, BATCH), :]
        iota = jax.lax.broadcasted_iota(jnp.int32, (BATCH, VOCAB), 1)
        oh_t = (iota == ids_t).astype(jnp.bfloat16)
        a = jnp.concatenate([oh_t, h_bf], axis=1)
        gates = jnp.dot(a, W_ref[:], preferred_element_type=jnp.float32)
        i = jax.nn.sigmoid(gates[:, 0 * HIDDEN:1 * HIDDEN])
        f = jax.nn.sigmoid(gates[:, 1 * HIDDEN:2 * HIDDEN])
        g = jnp.tanh(gates[:, 2 * HIDDEN:3 * HIDDEN])
        o = jax.nn.sigmoid(gates[:, 3 * HIDDEN:4 * HIDDEN])
        c_new = f * c + i * g
        h_new = o * jnp.tanh(c_new)
        return h_new.astype(jnp.bfloat16), c_new

    def step(k, carry):
        h_bf, c = carry
        for u in range(UNROLL):
            h_bf, c = substep(UNROLL * k + u, h_bf, c)
        return (h_bf, c)

    h0 = jnp.zeros((BATCH, HIDDEN), jnp.bfloat16)
    c0 = jnp.zeros((BATCH, HIDDEN), jnp.float32)
    h_last, _ = jax.lax.fori_loop(0, SEQ // UNROLL, step, (h0, c0))

    out_ref[:] = (jnp.dot(h_last.astype(jnp.float32), WfcT_ref[:],
                          preferred_element_type=jnp.float32) + bfc_ref[:])


def kernel(x, emb, W_ih, W_hh, b_ih, b_hh, W_fc, b_fc):
    x_col = x.T.reshape(SEQ * BATCH, 1).astype(jnp.int32)
    WihT = W_ih.T
    WhhT_bf = W_hh.T.astype(jnp.bfloat16)
    WfcT = W_fc.T
    bias = (b_ih + b_hh).reshape(1, 4 * HIDDEN)
    bfc = b_fc.reshape(1, VOCAB)

    return pl.pallas_call(
        _lstm_kernel,
        out_shape=jax.ShapeDtypeStruct((BATCH, VOCAB), jnp.float32),
        scratch_shapes=[
            pltpu.VMEM((VOCAB + HIDDEN, 4 * HIDDEN), jnp.bfloat16)],
    )(x_col, emb, WihT, WhhT_bf, bias, WfcT, bfc)
